# Initial kernel scaffold; baseline (speedup 1.0000x reference)
import jax
import jax.numpy as jnp
from jax.experimental import pallas as pl


def kernel(batch_input, table, W1, b1, W2, b2):
    raise NotImplementedError("write your pallas kernel here")


# speedup vs baseline: 12.9664x; 12.9664x over previous
import functools

import jax
import jax.numpy as jnp
from jax import lax
from jax.experimental import pallas as pl
from jax.experimental.pallas import tpu as pltpu
from jax.experimental.pallas import tpu_sc as plsc

B = 4096
SEQ = 200
D = 64
HIDDEN = 128
N_CLASSES = 100
HALF = SEQ // 2

NC = 2
NS = 16
NW = NC * NS
BPW = B // NW
LANES = 16
DCH = D // LANES


def _make_bagmean():
  mesh = plsc.VectorSubcoreMesh(core_axis_name="c", subcore_axis_name="s")

  @functools.partial(
      pl.kernel,
      mesh=mesh,
      compiler_params=pltpu.CompilerParams(use_tc_tiling_on_sc=False),
      out_type=jax.ShapeDtypeStruct((B, D), jnp.float32),
      scratch_types=[
          pltpu.VMEM((BPW, 2, HALF), jnp.int32),
          pltpu.VMEM((2, SEQ, D), jnp.float32),
          pltpu.VMEM((BPW, D), jnp.float32),
          pltpu.SemaphoreType.DMA,
          pltpu.SemaphoreType.DMA,
      ],
  )
  def bagmean(idx_hbm, table_hbm, out_hbm, idx_v, rows_v, out_v, sem0, sem1):
    wid = lax.axis_index("s") * NC + lax.axis_index("c")
    base = wid * BPW
    pltpu.sync_copy(idx_hbm.at[pl.ds(base, BPW)], idx_v)

    sems = (sem0, sem1)

    def issue(j, b):
      pltpu.async_copy(table_hbm.at[idx_v.at[j, 0]],
                       rows_v.at[b, pl.ds(0, HALF)], sems[b])
      pltpu.async_copy(table_hbm.at[idx_v.at[j, 1]],
                       rows_v.at[b, pl.ds(HALF, HALF)], sems[b])

    def wait_pair(j, b):
      pltpu.make_async_copy(table_hbm.at[idx_v.at[j, 0]],
                            rows_v.at[b, pl.ds(0, HALF)], sems[b]).wait()
      pltpu.make_async_copy(table_hbm.at[idx_v.at[j, 1]],
                            rows_v.at[b, pl.ds(HALF, HALF)], sems[b]).wait()

    def accumulate(j, b):
      def row_body(r, acc):
        return tuple(acc[c] + rows_v[b, r, pl.ds(c * LANES, LANES)]
                     for c in range(DCH))
      acc = lax.fori_loop(
          0, SEQ, row_body,
          tuple(jnp.zeros((LANES,), jnp.float32) for _ in range(DCH)),
          unroll=4)
      for c in range(DCH):
        out_v[j, pl.ds(c * LANES, LANES)] = acc[c] * (1.0 / SEQ)

    issue(0, 0)

    def pair_body(p, carry):
      j = 2 * p
      issue(j + 1, 1)
      wait_pair(j, 0)
      accumulate(j, 0)
      issue(lax.rem(j + 2, BPW), 0)
      wait_pair(j + 1, 1)
      accumulate(j + 1, 1)
      return carry

    lax.fori_loop(0, BPW // 2, pair_body, 0)
    wait_pair(0, 0)

    pltpu.sync_copy(out_v, out_hbm.at[pl.ds(base, BPW)])

  return bagmean


def _mlp_body(x_ref, w1t_ref, b1_ref, w2t_ref, b2_ref, out_ref):
  x = x_ref[...]
  h = jnp.dot(x, w1t_ref[...], preferred_element_type=jnp.float32)
  h = jnp.maximum(h + b1_ref[...], 0.0)
  out_ref[...] = (
      jnp.dot(h, w2t_ref[...], preferred_element_type=jnp.float32)
      + b2_ref[...])


def _mlp(x, W1, b1, W2, b2):
  blk = 512
  grid = (B // blk,)
  return pl.pallas_call(
      _mlp_body,
      grid=grid,
      in_specs=[
          pl.BlockSpec((blk, D), lambda i: (i, 0)),
          pl.BlockSpec((D, HIDDEN), lambda i: (0, 0)),
          pl.BlockSpec((1, HIDDEN), lambda i: (0, 0)),
          pl.BlockSpec((HIDDEN, N_CLASSES), lambda i: (0, 0)),
          pl.BlockSpec((1, N_CLASSES), lambda i: (0, 0)),
      ],
      out_specs=pl.BlockSpec((blk, N_CLASSES), lambda i: (i, 0)),
      out_shape=jax.ShapeDtypeStruct((B, N_CLASSES), jnp.float32),
  )(x, W1.T, b1.reshape(1, HIDDEN), W2.T, b2.reshape(1, N_CLASSES))


def kernel(batch_input, table, W1, b1, W2, b2):
  idx3 = batch_input.reshape(B, 2, HALF)
  x = _make_bagmean()(idx3, table)
  return _mlp(x, W1, b1, W2, b2)
